# baseline (speedup 1.0000x reference)
import jax
import jax.numpy as jnp
from jax import lax
from jax.experimental import pallas as pl
from jax.experimental.pallas import tpu as pltpu


def _make_body(H, Tc, B, n_chunks):
    G = 4 * H
    NT = G // 256
    EPT = B // 4
    BANK = NT * EPT

    def body(x_ref, wih0_ref, wih1_ref, whh0_ref, whh1_ref, b_ref,
             h_ref, c_ref, xbuf, g0buf, g1buf):
        c_idx = pl.program_id(1)
        f32 = jnp.float32

        def wtile(wt_ref, n):
            return wt_ref[n]

        def big_gates(get_chunk, wt_ref, dst, layer):
            bias = b_ref[layer]
            M = Tc * B
            MC = 256
            nmc = M // MC
            for mxu in (0, 1):
                pltpu.matmul_push_rhs(wtile(wt_ref, 2 * mxu),
                                      staging_register=0, mxu_index=mxu)
                pltpu.matmul_push_rhs(wtile(wt_ref, 2 * mxu + 1),
                                      staging_register=1, mxu_index=mxu)
            for mxu in (0, 1):
                for ti in range(2):
                    n = 2 * mxu + ti
                    col = slice(n * 256, (n + 1) * 256)
                    btile = bias[:, col]
                    for j in range(nmc):
                        addr = 64 + (j % 3) * 64
                        pltpu.matmul_acc_lhs(
                            addr, get_chunk(j), mxu_index=mxu,
                            load_staged_rhs=(ti if j == 0 else None))
                        if j >= 2:
                            pa = 64 + ((j - 2) % 3) * 64
                            g = pltpu.matmul_pop(pa, (MC, 256), f32,
                                                 mxu_index=mxu)
                            dst[pl.ds((j - 2) * MC, MC), col] = g + btile
                    for j in (nmc - 2, nmc - 1):
                        pa = 64 + (j % 3) * 64
                        g = pltpu.matmul_pop(pa, (MC, 256), f32,
                                             mxu_index=mxu)
                        dst[pl.ds(j * MC, MC), col] = g + btile

        def rec_step(mxu, wt_ref, gbuf, row, bank_pop, bank_acc, c,
                     store_x):
            pb = bank_pop * BANK
            ab = bank_acc * BANK
            p = [pltpu.matmul_pop(pb + EPT * n, (B, 256), f32,
                                  mxu_index=mxu) for n in range(NT)]
            gi = gbuf[pl.ds(row, B), 0:256] + p[0]
            gf = gbuf[pl.ds(row, B), 256:512] + p[1]
            go = gbuf[pl.ds(row, B), 512:768] + p[2]
            gg = gbuf[pl.ds(row, B), 768:1024] + p[3]
            pltpu.matmul_push_rhs(wtile(wt_ref, 0), staging_register=0,
                                  mxu_index=mxu)
            pltpu.matmul_push_rhs(wtile(wt_ref, 1), staging_register=1,
                                  mxu_index=mxu)
            si = 0.5 + 0.5 * jnp.tanh(gi)
            sf = 0.5 + 0.5 * jnp.tanh(gf)
            so = 0.5 + 0.5 * jnp.tanh(go)
            tg = jnp.tanh(gg)
            cn = sf * c + si * tg
            hn = so * jnp.tanh(cn)
            hb = hn.astype(jnp.bfloat16)
            if store_x:
                xbuf[pl.ds(row, B), :] = hb
            pltpu.matmul_acc_lhs(ab, hb, mxu_index=mxu, load_staged_rhs=0)
            pltpu.matmul_push_rhs(wtile(wt_ref, 2), staging_register=0,
                                  mxu_index=mxu)
            pltpu.matmul_acc_lhs(ab + EPT, hb, mxu_index=mxu,
                                 load_staged_rhs=1)
            pltpu.matmul_push_rhs(wtile(wt_ref, 3), staging_register=1,
                                  mxu_index=mxu)
            pltpu.matmul_acc_lhs(ab + 2 * EPT, hb, mxu_index=mxu,
                                 load_staged_rhs=0)
            pltpu.matmul_acc_lhs(ab + 3 * EPT, hb, mxu_index=mxu,
                                 load_staged_rhs=1)
            return hn, cn

        def step_l0(s2, carry):
            h0, c0 = carry
            row = pl.multiple_of(2 * s2 * B, B)
            h0, c0 = rec_step(0, whh0_ref, g0buf, row, 0, 1, c0, True)
            h0, c0 = rec_step(0, whh0_ref, g0buf, row + B, 1, 0, c0, True)
            return h0, c0

        def step_fused(s2, carry):
            h0, c0, h1, c1 = carry
            row = pl.multiple_of(2 * s2 * B, B)
            h0, c0 = rec_step(0, whh0_ref, g0buf, row, 0, 1, c0, True)
            h1, c1 = rec_step(1, whh1_ref, g1buf, row, 0, 1, c1, False)
            h0, c0 = rec_step(0, whh0_ref, g0buf, row + B, 1, 0, c0, True)
            h1, c1 = rec_step(1, whh1_ref, g1buf, row + B, 1, 0, c1, False)
            return h0, c0, h1, c1

        def step_l1(s2, carry):
            h1, c1 = carry
            row = pl.multiple_of(2 * s2 * B, B)
            h1, c1 = rec_step(1, whh1_ref, g1buf, row, 0, 1, c1, False)
            h1, c1 = rec_step(1, whh1_ref, g1buf, row + B, 1, 0, c1, False)
            return h1, c1

        @pl.when(c_idx == 0)
        def _():
            pltpu.matmul_pop(0, (8 * BANK, 256), f32, mxu_index=0)
            pltpu.matmul_pop(0, (8 * BANK, 256), f32, mxu_index=1)

        def x_chunk(j):
            return x_ref[pl.ds(j * (256 // B), 256 // B), :, :].reshape(256, H)

        def xbuf_chunk(j):
            return xbuf[pl.ds(j * 256, 256), :]

        big_gates(x_chunk, wih0_ref, g0buf, 0)

        @pl.when(c_idx == 0)
        def _():
            z = jnp.zeros((B, H), f32)
            h0, c0 = lax.fori_loop(0, Tc // 2, step_l0, (z, z))
            h_ref[0], c_ref[0] = h0, c0
            h_ref[1] = jnp.zeros((B, H), f32)
            c_ref[1] = jnp.zeros((B, H), f32)

        @pl.when(c_idx > 0)
        def _():
            big_gates(xbuf_chunk, wih1_ref, g1buf, 1)
            carry = (h_ref[0], c_ref[0], h_ref[1], c_ref[1])
            h0, c0, h1, c1 = lax.fori_loop(0, Tc // 2, step_fused, carry,
                                           unroll=4)
            h_ref[0], c_ref[0] = h0, c0
            h_ref[1], c_ref[1] = h1, c1

        @pl.when(c_idx == n_chunks - 1)
        def _():
            big_gates(xbuf_chunk, wih1_ref, g1buf, 1)
            carry = (h_ref[1], c_ref[1])
            h1, c1 = lax.fori_loop(0, Tc // 2, step_l1, carry)
            h_ref[1], c_ref[1] = h1, c1
            pltpu.matmul_pop(0, (4 * BANK, 256), f32, mxu_index=0)
            pltpu.matmul_pop(0, (4 * BANK, 256), f32, mxu_index=1)

    return body


def kernel(in_seq, w_ih0, w_ihr, w_hh, b):
    B, T, D = in_seq.shape
    L, H, G = w_hh.shape
    assert L == 2 and G == 4 * H and H == 256 and B % 8 == 0
    Tc = 64 if T % 64 == 0 else T
    n_chunks = T // Tc

    x = jnp.transpose(in_seq, (1, 0, 2)).astype(jnp.bfloat16)
    half = jnp.concatenate([jnp.full((3 * H,), 0.5, jnp.float32),
                            jnp.ones((H,), jnp.float32)])

    def tiles(w):
        wb = (w * half).astype(jnp.bfloat16)
        return jnp.transpose(wb.reshape(H, G // 256, 256), (1, 0, 2))

    wih0 = tiles(w_ih0)
    wih1 = tiles(w_ihr[0])
    whh0 = tiles(w_hh[0])
    whh1 = tiles(w_hh[1])
    b = b * half

    Bh = B // 2
    body = _make_body(H, Tc, Bh, n_chunks)

    out_shapes = (
        jax.ShapeDtypeStruct((L, B, H), jnp.float32),
        jax.ShapeDtypeStruct((L, B, H), jnp.float32),
    )
    h_out, c_out = pl.pallas_call(
        body,
        out_shape=out_shapes,
        grid=(2, n_chunks),
        in_specs=[
            pl.BlockSpec((Tc, Bh, D), lambda k, c: (c, k, 0)),
            pl.BlockSpec((G // 256, H, 256), lambda k, c: (0, 0, 0)),
            pl.BlockSpec((G // 256, H, 256), lambda k, c: (0, 0, 0)),
            pl.BlockSpec((G // 256, H, 256), lambda k, c: (0, 0, 0)),
            pl.BlockSpec((G // 256, H, 256), lambda k, c: (0, 0, 0)),
            pl.BlockSpec((L, 1, G), lambda k, c: (0, 0, 0)),
        ],
        out_specs=(
            pl.BlockSpec((L, Bh, H), lambda k, c: (0, k, 0)),
            pl.BlockSpec((L, Bh, H), lambda k, c: (0, k, 0)),
        ),
        scratch_shapes=[
            pltpu.VMEM((Tc * Bh, H), jnp.bfloat16),
            pltpu.VMEM((Tc * Bh, G), jnp.float32),
            pltpu.VMEM((Tc * Bh, G), jnp.float32),
        ],
        compiler_params=pltpu.CompilerParams(
            dimension_semantics=("parallel", "arbitrary"),
            vmem_limit_bytes=48 * 2 ** 20),
    )(x, wih0, wih1, whh0, whh1, b)

    return h_out, c_out


# speedup vs baseline: 1.5834x; 1.5834x over previous
import jax
import jax.numpy as jnp
from jax import lax
from jax.experimental import pallas as pl
from jax.experimental.pallas import tpu as pltpu


def _make_body(H, Tc, B, n_chunks, unroll):
    G = 4 * H

    def gate_math(g, c):
        sig = 0.5 + 0.5 * jnp.tanh(g[:, :3 * H])
        gg = jnp.tanh(g[:, 3 * H:])
        c_new = sig[:, H:2 * H] * c + sig[:, :H] * gg
        h_new = sig[:, 2 * H:3 * H] * jnp.tanh(c_new)
        return h_new, c_new

    def body(x_ref, wih0_ref, wih1_ref, whh0_ref, whh1_ref, b_ref,
             h_ref, c_ref, xbuf, g0buf, g1buf):
        c_idx = pl.program_id(0)

        w0 = whh0_ref[...]
        w1 = whh1_ref[...]

        def hpart(h, w):
            return jnp.dot(h.astype(jnp.bfloat16), w,
                           preferred_element_type=jnp.float32)

        def step_l0(s, carry):
            _, c0, p0 = carry
            row = pl.multiple_of(s * B, B)
            g0 = g0buf[pl.ds(row, B), :] + p0
            h0n, c0n = gate_math(g0, c0)
            xbuf[pl.ds(row, B), :] = h0n.astype(jnp.bfloat16)
            return h0n, c0n, hpart(h0n, w0)

        def step_fused(s, carry):
            _, c0, _, c1, p0, p1 = carry
            row = pl.multiple_of(s * B, B)
            g0 = g0buf[pl.ds(row, B), :] + p0
            h0n, c0n = gate_math(g0, c0)
            xbuf[pl.ds(row, B), :] = h0n.astype(jnp.bfloat16)
            g1 = g1buf[pl.ds(row, B), :] + p1
            h1n, c1n = gate_math(g1, c1)
            return h0n, c0n, h1n, c1n, hpart(h0n, w0), hpart(h1n, w1)

        def step_l1(s, carry):
            _, c1, p1 = carry
            row = pl.multiple_of(s * B, B)
            g1 = g1buf[pl.ds(row, B), :] + p1
            h1n, c1n = gate_math(g1, c1)
            return h1n, c1n, hpart(h1n, w1)

        g0buf[...] = jnp.dot(x_ref[...], wih0_ref[...],
                             preferred_element_type=jnp.float32) + b_ref[0]

        @pl.when(c_idx == 0)
        def _():
            z = jnp.zeros((B, H), jnp.float32)
            zg = jnp.zeros((B, G), jnp.float32)
            h0, c0, _ = lax.fori_loop(0, Tc, step_l0, (z, z, zg),
                                      unroll=unroll)
            h_ref[0], c_ref[0] = h0, c0
            h_ref[1] = jnp.zeros((B, H), jnp.float32)
            c_ref[1] = jnp.zeros((B, H), jnp.float32)

        @pl.when(c_idx > 0)
        def _():
            g1buf[...] = jnp.dot(xbuf[...], wih1_ref[...],
                                 preferred_element_type=jnp.float32) + b_ref[1]
            h0, c0 = h_ref[0], c_ref[0]
            h1, c1 = h_ref[1], c_ref[1]
            carry = (h0, c0, h1, c1, hpart(h0, w0), hpart(h1, w1))
            h0, c0, h1, c1, _, _ = lax.fori_loop(0, Tc, step_fused, carry,
                                                 unroll=unroll)
            h_ref[0], c_ref[0] = h0, c0
            h_ref[1], c_ref[1] = h1, c1

        @pl.when(c_idx == n_chunks - 1)
        def _():
            g1buf[...] = jnp.dot(xbuf[...], wih1_ref[...],
                                 preferred_element_type=jnp.float32) + b_ref[1]
            h1, c1 = h_ref[1], c_ref[1]
            carry = (h1, c1, hpart(h1, w1))
            h1, c1, _ = lax.fori_loop(0, Tc, step_l1, carry, unroll=unroll)
            h_ref[1], c_ref[1] = h1, c1

    return body


def kernel(in_seq, w_ih0, w_ihr, w_hh, b):
    B, T, D = in_seq.shape
    L, H, G = w_hh.shape
    assert L == 2 and G == 4 * H and B % 8 == 0
    Tc = 64 if T % 64 == 0 else T
    n_chunks = T // Tc

    x = jnp.transpose(in_seq, (1, 0, 2)).reshape(T * B, D).astype(jnp.bfloat16)
    half = jnp.concatenate([jnp.full((3 * H,), 0.5, jnp.float32),
                            jnp.ones((H,), jnp.float32)])
    wih0 = (w_ih0 * half).astype(jnp.bfloat16)
    wih1 = (w_ihr[0] * half).astype(jnp.bfloat16)
    whh0 = (w_hh[0] * half).astype(jnp.bfloat16)
    whh1 = (w_hh[1] * half).astype(jnp.bfloat16)
    b = b * half

    body = _make_body(H, Tc, B, n_chunks, unroll=8)

    out_shapes = (
        jax.ShapeDtypeStruct((L, B, H), jnp.float32),
        jax.ShapeDtypeStruct((L, B, H), jnp.float32),
    )
    h_out, c_out = pl.pallas_call(
        body,
        out_shape=out_shapes,
        grid=(n_chunks,),
        in_specs=[
            pl.BlockSpec((Tc * B, D), lambda c: (c, 0)),
            pl.BlockSpec((D, G), lambda c: (0, 0)),
            pl.BlockSpec((H, G), lambda c: (0, 0)),
            pl.BlockSpec((H, G), lambda c: (0, 0)),
            pl.BlockSpec((H, G), lambda c: (0, 0)),
            pl.BlockSpec((L, 1, G), lambda c: (0, 0, 0)),
        ],
        out_specs=(
            pl.BlockSpec((L, B, H), lambda c: (0, 0, 0)),
            pl.BlockSpec((L, B, H), lambda c: (0, 0, 0)),
        ),
        scratch_shapes=[
            pltpu.VMEM((Tc * B, H), jnp.bfloat16),
            pltpu.VMEM((Tc * B, G), jnp.float32),
            pltpu.VMEM((Tc * B, G), jnp.float32),
        ],
        compiler_params=pltpu.CompilerParams(
            dimension_semantics=("arbitrary",),
            vmem_limit_bytes=48 * 2 ** 20),
    )(x, wih0, wih1, whh0, whh1, b)

    return h_out, c_out


# speedup vs baseline: 1.6801x; 1.0610x over previous
import jax
import jax.numpy as jnp
from jax import lax
from jax.experimental import pallas as pl
from jax.experimental.pallas import tpu as pltpu


def _make_body(H, Tc, B, n_chunks, unroll):
    G = 4 * H

    def gate_math(g, c):
        sig = jax.nn.sigmoid(g[:, :3 * H])
        gg = jnp.tanh(g[:, 3 * H:])
        c_new = sig[:, H:2 * H] * c + sig[:, :H] * gg
        h_new = sig[:, 2 * H:3 * H] * jnp.tanh(c_new)
        return h_new, c_new

    def body(x_ref, wih0_ref, wih1_ref, whh0_ref, whh1_ref, b_ref,
             h_ref, c_ref, xbuf, g0buf, g1buf):
        c_idx = pl.program_id(0)

        w0 = whh0_ref[...]
        w1 = whh1_ref[...]

        def hpart(h, w):
            return jnp.dot(h.astype(jnp.bfloat16), w,
                           preferred_element_type=jnp.float32)

        def step_l0(s, carry):
            _, c0, p0 = carry
            row = pl.multiple_of(s * B, B)
            g0 = g0buf[pl.ds(row, B), :] + p0
            h0n, c0n = gate_math(g0, c0)
            xbuf[pl.ds(row, B), :] = h0n.astype(jnp.bfloat16)
            return h0n, c0n, hpart(h0n, w0)

        def step_fused(s, carry):
            _, c0, _, c1, p0, p1 = carry
            row = pl.multiple_of(s * B, B)
            g0 = g0buf[pl.ds(row, B), :] + p0
            h0n, c0n = gate_math(g0, c0)
            xbuf[pl.ds(row, B), :] = h0n.astype(jnp.bfloat16)
            g1 = g1buf[pl.ds(row, B), :] + p1
            h1n, c1n = gate_math(g1, c1)
            return h0n, c0n, h1n, c1n, hpart(h0n, w0), hpart(h1n, w1)

        def step_l1(s, carry):
            _, c1, p1 = carry
            row = pl.multiple_of(s * B, B)
            g1 = g1buf[pl.ds(row, B), :] + p1
            h1n, c1n = gate_math(g1, c1)
            return h1n, c1n, hpart(h1n, w1)

        g0buf[...] = jnp.dot(x_ref[...], wih0_ref[...],
                             preferred_element_type=jnp.float32) + b_ref[0]

        @pl.when(c_idx == 0)
        def _():
            z = jnp.zeros((B, H), jnp.float32)
            zg = jnp.zeros((B, G), jnp.float32)
            h0, c0, _ = lax.fori_loop(0, Tc, step_l0, (z, z, zg),
                                      unroll=unroll)
            h_ref[0], c_ref[0] = h0, c0
            h_ref[1] = jnp.zeros((B, H), jnp.float32)
            c_ref[1] = jnp.zeros((B, H), jnp.float32)

        @pl.when(c_idx > 0)
        def _():
            g1buf[...] = jnp.dot(xbuf[...], wih1_ref[...],
                                 preferred_element_type=jnp.float32) + b_ref[1]
            h0, c0 = h_ref[0], c_ref[0]
            h1, c1 = h_ref[1], c_ref[1]
            carry = (h0, c0, h1, c1, hpart(h0, w0), hpart(h1, w1))
            h0, c0, h1, c1, _, _ = lax.fori_loop(0, Tc, step_fused, carry,
                                                 unroll=unroll)
            h_ref[0], c_ref[0] = h0, c0
            h_ref[1], c_ref[1] = h1, c1

        @pl.when(c_idx == n_chunks - 1)
        def _():
            g1buf[...] = jnp.dot(xbuf[...], wih1_ref[...],
                                 preferred_element_type=jnp.float32) + b_ref[1]
            h1, c1 = h_ref[1], c_ref[1]
            carry = (h1, c1, hpart(h1, w1))
            h1, c1, _ = lax.fori_loop(0, Tc, step_l1, carry, unroll=unroll)
            h_ref[1], c_ref[1] = h1, c1

    return body


def kernel(in_seq, w_ih0, w_ihr, w_hh, b):
    B, T, D = in_seq.shape
    L, H, G = w_hh.shape
    assert L == 2 and G == 4 * H and B % 8 == 0
    Tc = 32 if T % 32 == 0 else T
    n_chunks = T // Tc

    x = jnp.transpose(in_seq, (1, 0, 2)).reshape(T * B, D).astype(jnp.bfloat16)
    wih0 = w_ih0.astype(jnp.bfloat16)
    wih1 = w_ihr[0].astype(jnp.bfloat16)
    whh0 = w_hh[0].astype(jnp.bfloat16)
    whh1 = w_hh[1].astype(jnp.bfloat16)

    body = _make_body(H, Tc, B, n_chunks, unroll=8)

    out_shapes = (
        jax.ShapeDtypeStruct((L, B, H), jnp.float32),
        jax.ShapeDtypeStruct((L, B, H), jnp.float32),
    )
    h_out, c_out = pl.pallas_call(
        body,
        out_shape=out_shapes,
        grid=(n_chunks,),
        in_specs=[
            pl.BlockSpec((Tc * B, D), lambda c: (c, 0)),
            pl.BlockSpec((D, G), lambda c: (0, 0)),
            pl.BlockSpec((H, G), lambda c: (0, 0)),
            pl.BlockSpec((H, G), lambda c: (0, 0)),
            pl.BlockSpec((H, G), lambda c: (0, 0)),
            pl.BlockSpec((L, 1, G), lambda c: (0, 0, 0)),
        ],
        out_specs=(
            pl.BlockSpec((L, B, H), lambda c: (0, 0, 0)),
            pl.BlockSpec((L, B, H), lambda c: (0, 0, 0)),
        ),
        scratch_shapes=[
            pltpu.VMEM((Tc * B, H), jnp.bfloat16),
            pltpu.VMEM((Tc * B, G), jnp.float32),
            pltpu.VMEM((Tc * B, G), jnp.float32),
        ],
        compiler_params=pltpu.CompilerParams(
            dimension_semantics=("arbitrary",),
            vmem_limit_bytes=48 * 2 ** 20),
    )(x, wih0, wih1, whh0, whh1, b)

    return h_out, c_out


# speedup vs baseline: 2.0058x; 1.1939x over previous
import jax
import jax.numpy as jnp
from jax import lax
from jax.experimental import pallas as pl
from jax.experimental.pallas import tpu as pltpu


def _make_body(H, Tc, B, n_chunks, unroll):
    G = 4 * H

    def gate_math(g, c):
        sig = jax.nn.sigmoid(g[:, :3 * H])
        gg = jnp.tanh(g[:, 3 * H:])
        c_new = sig[:, H:2 * H] * c + sig[:, :H] * gg
        h_new = sig[:, 2 * H:3 * H] * jnp.tanh(c_new)
        return h_new, c_new

    def body(x_ref, wih0_ref, wih1_ref, whh0_ref, whh1_ref, b_ref,
             h_ref, c_ref, xbuf, g0buf, g1buf):
        c_idx = pl.program_id(0)

        w0 = whh0_ref[...]
        w1 = whh1_ref[...]

        def hpart(h, w):
            return jnp.dot(h.astype(jnp.bfloat16), w,
                           preferred_element_type=jnp.float32)

        def step_l0(s, carry):
            _, c0, p0 = carry
            row = pl.multiple_of(s * B, B)
            g0 = g0buf[pl.ds(row, B), :] + p0
            h0n, c0n = gate_math(g0, c0)
            xbuf[pl.ds(row, B), :] = h0n.astype(jnp.bfloat16)
            return h0n, c0n, hpart(h0n, w0)

        def step_fused(s, carry):
            _, c0, _, c1, p0, p1 = carry
            row = pl.multiple_of(s * B, B)
            g0 = g0buf[pl.ds(row, B), :] + p0
            h0n, c0n = gate_math(g0, c0)
            xbuf[pl.ds(row, B), :] = h0n.astype(jnp.bfloat16)
            g1 = g1buf[pl.ds(row, B), :] + p1
            h1n, c1n = gate_math(g1, c1)
            return h0n, c0n, h1n, c1n, hpart(h0n, w0), hpart(h1n, w1)

        def step_l1(s, carry):
            _, c1, p1 = carry
            row = pl.multiple_of(s * B, B)
            g1 = g1buf[pl.ds(row, B), :] + p1
            h1n, c1n = gate_math(g1, c1)
            return h1n, c1n, hpart(h1n, w1)

        xt = jnp.transpose(x_ref[...], (1, 0, 2)).reshape(Tc * B, H)
        g0buf[...] = jnp.dot(xt.astype(jnp.bfloat16), wih0_ref[...],
                             preferred_element_type=jnp.float32) + b_ref[0]

        @pl.when(c_idx == 0)
        def _():
            z = jnp.zeros((B, H), jnp.float32)
            zg = jnp.zeros((B, G), jnp.float32)
            h0, c0, _ = lax.fori_loop(0, Tc, step_l0, (z, z, zg),
                                      unroll=unroll)
            h_ref[0], c_ref[0] = h0, c0
            h_ref[1] = jnp.zeros((B, H), jnp.float32)
            c_ref[1] = jnp.zeros((B, H), jnp.float32)

        @pl.when(c_idx > 0)
        def _():
            g1buf[...] = jnp.dot(xbuf[...], wih1_ref[...],
                                 preferred_element_type=jnp.float32) + b_ref[1]
            h0, c0 = h_ref[0], c_ref[0]
            h1, c1 = h_ref[1], c_ref[1]
            carry = (h0, c0, h1, c1, hpart(h0, w0), hpart(h1, w1))
            h0, c0, h1, c1, _, _ = lax.fori_loop(0, Tc, step_fused, carry,
                                                 unroll=unroll)
            h_ref[0], c_ref[0] = h0, c0
            h_ref[1], c_ref[1] = h1, c1

        @pl.when(c_idx == n_chunks - 1)
        def _():
            g1buf[...] = jnp.dot(xbuf[...], wih1_ref[...],
                                 preferred_element_type=jnp.float32) + b_ref[1]
            h1, c1 = h_ref[1], c_ref[1]
            carry = (h1, c1, hpart(h1, w1))
            h1, c1, _ = lax.fori_loop(0, Tc, step_l1, carry, unroll=unroll)
            h_ref[1], c_ref[1] = h1, c1

    return body


def kernel(in_seq, w_ih0, w_ihr, w_hh, b):
    B, T, D = in_seq.shape
    L, H, G = w_hh.shape
    assert L == 2 and G == 4 * H and B % 8 == 0
    Tc = 32 if T % 32 == 0 else T
    n_chunks = T // Tc

    x = in_seq
    wih0 = w_ih0.astype(jnp.bfloat16)
    wih1 = w_ihr[0].astype(jnp.bfloat16)
    whh0 = w_hh[0].astype(jnp.bfloat16)
    whh1 = w_hh[1].astype(jnp.bfloat16)

    body = _make_body(H, Tc, B, n_chunks, unroll=8)

    out_shapes = (
        jax.ShapeDtypeStruct((L, B, H), jnp.float32),
        jax.ShapeDtypeStruct((L, B, H), jnp.float32),
    )
    h_out, c_out = pl.pallas_call(
        body,
        out_shape=out_shapes,
        grid=(n_chunks,),
        in_specs=[
            pl.BlockSpec((B, Tc, D), lambda c: (0, c, 0)),
            pl.BlockSpec((D, G), lambda c: (0, 0)),
            pl.BlockSpec((H, G), lambda c: (0, 0)),
            pl.BlockSpec((H, G), lambda c: (0, 0)),
            pl.BlockSpec((H, G), lambda c: (0, 0)),
            pl.BlockSpec((L, 1, G), lambda c: (0, 0, 0)),
        ],
        out_specs=(
            pl.BlockSpec((L, B, H), lambda c: (0, 0, 0)),
            pl.BlockSpec((L, B, H), lambda c: (0, 0, 0)),
        ),
        scratch_shapes=[
            pltpu.VMEM((Tc * B, H), jnp.bfloat16),
            pltpu.VMEM((Tc * B, G), jnp.float32),
            pltpu.VMEM((Tc * B, G), jnp.float32),
        ],
        compiler_params=pltpu.CompilerParams(
            dimension_semantics=("arbitrary",),
            vmem_limit_bytes=48 * 2 ** 20),
    )(x, wih0, wih1, whh0, whh1, b)

    return h_out, c_out
